# baseline (speedup 1.0000x reference)
import functools
import jax
import jax.numpy as jnp
from jax import lax
from jax.experimental import pallas as pl
from jax.experimental.pallas import tpu as pltpu
from jax.experimental.pallas import tpu_sc as plsc

B, S, F = 4, 2048, 4096
FW = 128
CH = 256
NG = FW // 16
F_PER_W = F // 8
NSTRIP = F_PER_W // FW
NCHUNK = S // CH


def _sc_body(x_hbm, o_hbm, buf):
    cid = lax.axis_index("c")
    sid = lax.axis_index("s")
    wid = sid * 2 + cid
    b = wid // 8
    f_base = (wid % 8) * F_PER_W

    def do_strip(k, tok):
        f0 = f_base + k * FW

        def do_chunk(ci, carries):
            s0 = ci * CH
            pltpu.sync_copy(x_hbm.at[b, pl.ds(s0, CH), pl.ds(f0, FW)], buf)

            def s_body(si, carries):
                out = []
                for g in range(NG):
                    v = buf[si, pl.ds(g * 16, 16)]
                    nc = carries[g] + v
                    buf[si, pl.ds(g * 16, 16)] = nc
                    out.append(nc)
                return tuple(out)

            carries = lax.fori_loop(0, CH, s_body, carries)
            pltpu.sync_copy(buf, o_hbm.at[b, pl.ds(s0, CH), pl.ds(f0, FW)])
            return carries

        zero = jnp.zeros((16,), jnp.float32)
        lax.fori_loop(0, NCHUNK, do_chunk, tuple(zero for _ in range(NG)))
        return tok

    lax.fori_loop(0, NSTRIP, do_strip, 0)


def kernel(x):
    mesh = plsc.VectorSubcoreMesh(core_axis_name="c", subcore_axis_name="s")
    kfn = pl.kernel(
        _sc_body,
        mesh=mesh,
        out_type=jax.ShapeDtypeStruct((B, S, F), jnp.float32),
        scratch_types=[pltpu.VMEM((CH, FW), jnp.float32)],
    )
    return kfn(x)


# speedup vs baseline: 1.4879x; 1.4879x over previous
import jax
import jax.numpy as jnp
from jax import lax
from jax.experimental import pallas as pl
from jax.experimental.pallas import tpu as pltpu
from jax.experimental.pallas import tpu_sc as plsc

B, S, F = 4, 2048, 4096
FW = 128
CH = 128
NG = FW // 16
F_PER_W = F // 8
NSTRIP = F_PER_W // FW
NCHUNK = S // CH
T = NSTRIP * NCHUNK


def _sc_body(x_hbm, o_hbm, in0, in1, out0, out1, isem0, isem1, osem0, osem1):
    cid = lax.axis_index("c")
    sid = lax.axis_index("s")
    wid = sid * 2 + cid
    b = wid // 8
    f_base = (wid % 8) * F_PER_W

    ins = (in0, in1)
    outs = (out0, out1)
    isems = (isem0, isem1)
    osems = (osem0, osem1)

    def src_at(t):
        k = t // NCHUNK
        ci = lax.rem(t, NCHUNK)
        s0 = ci * CH
        f0 = f_base + k * FW
        return s0, f0

    def start_in(slot, t):
        s0, f0 = src_at(t)
        pltpu.async_copy(
            x_hbm.at[b, pl.ds(s0, CH), pl.ds(f0, FW)], ins[slot], isems[slot]
        )

    def start_out(slot, t):
        s0, f0 = src_at(t)
        pltpu.async_copy(
            outs[slot], o_hbm.at[b, pl.ds(s0, CH), pl.ds(f0, FW)], osems[slot]
        )

    def wait_in(slot):
        pltpu.make_async_copy(x_hbm.at[b, pl.ds(0, CH), pl.ds(0, FW)],
                              ins[slot], isems[slot]).wait()

    def wait_out(slot):
        pltpu.make_async_copy(outs[slot],
                              o_hbm.at[b, pl.ds(0, CH), pl.ds(0, FW)],
                              osems[slot]).wait()

    start_in(0, 0)
    start_in(1, 1)

    def pair_body(i2, carries):
        for par in range(2):
            t = i2 * 2 + par
            ci = lax.rem(t, NCHUNK)
            wait_in(par)

            @pl.when(t >= 2)
            def _():
                wait_out(par)

            zero = jnp.zeros((16,), jnp.float32)
            carries = tuple(
                jnp.where(ci == 0, zero, c) for c in carries
            )

            def s_body(si, carries):
                nxt = []
                for g in range(NG):
                    v = ins[par][si, pl.ds(g * 16, 16)]
                    nc = carries[g] + v
                    outs[par][si, pl.ds(g * 16, 16)] = nc
                    nxt.append(nc)
                return tuple(nxt)

            carries = lax.fori_loop(0, CH, s_body, carries)
            start_out(par, t)

            @pl.when(t + 2 < T)
            def _():
                start_in(par, t + 2)
        return carries

    zero = jnp.zeros((16,), jnp.float32)
    lax.fori_loop(0, T // 2, pair_body, tuple(zero for _ in range(NG)))
    wait_out(0)
    wait_out(1)


def kernel(x):
    mesh = plsc.VectorSubcoreMesh(core_axis_name="c", subcore_axis_name="s")
    kfn = pl.kernel(
        _sc_body,
        mesh=mesh,
        out_type=jax.ShapeDtypeStruct((B, S, F), jnp.float32),
        scratch_types=[
            pltpu.VMEM((CH, FW), jnp.float32),
            pltpu.VMEM((CH, FW), jnp.float32),
            pltpu.VMEM((CH, FW), jnp.float32),
            pltpu.VMEM((CH, FW), jnp.float32),
            pltpu.SemaphoreType.DMA,
            pltpu.SemaphoreType.DMA,
            pltpu.SemaphoreType.DMA,
            pltpu.SemaphoreType.DMA,
        ],
    )
    return kfn(x)
